# baseline (speedup 1.0000x reference)
import functools

import jax
import jax.numpy as jnp
from jax import lax
from jax.experimental import pallas as pl
from jax.experimental.pallas import tpu as pltpu
from jax.experimental.pallas import tpu_sc as plsc

_DP = 16


def _gather_call(V, B):
    info = plsc.get_sparse_core_info()
    NC, NS = info.num_cores, info.num_subcores
    NW = NC * NS
    assert B % (8 * NW) == 0
    b_per_w = B // NW
    mesh = plsc.VectorSubcoreMesh(core_axis_name="c", subcore_axis_name="s")

    chunk = 128
    n_chunks = b_per_w // chunk
    assert n_chunks * chunk == b_per_w

    @functools.partial(
        pl.kernel,
        mesh=mesh,
        out_type=jax.ShapeDtypeStruct((B, _DP), jnp.float32),
        scratch_types=[
            pltpu.VMEM((b_per_w,), jnp.int32),
            pltpu.VMEM((b_per_w, _DP), jnp.float32),
            pltpu.SemaphoreType.DMA,
            pltpu.SemaphoreType.DMA,
        ],
        compiler_params=pltpu.CompilerParams(use_tc_tiling_on_sc=False),
    )
    def k(table_hbm, idx_hbm, out_hbm, idx_v, rows_v, sem_g, sem_w):
        wid = lax.axis_index("s") * NC + lax.axis_index("c")
        base = wid * b_per_w
        pltpu.sync_copy(idx_hbm.at[pl.ds(base, b_per_w)], idx_v)
        gathers = [
            pltpu.async_copy(
                table_hbm.at[idx_v.at[pl.ds(j * chunk, chunk)]],
                rows_v.at[pl.ds(j * chunk, chunk)],
                sem_g,
            )
            for j in range(n_chunks)
        ]
        writes = []
        for j in range(n_chunks):
            gathers[j].wait()
            writes.append(
                pltpu.async_copy(
                    rows_v.at[pl.ds(j * chunk, chunk)],
                    out_hbm.at[pl.ds(base + j * chunk, chunk)],
                    sem_w,
                )
            )
        for w in writes:
            w.wait()

    return k


def _prep_body(se3_ref, out_ref):
    x = se3_ref[...]
    xp = jnp.pad(x, ((0, 0), (0, _DP - 12)))
    z = xp.reshape(xp.shape[0] // 8, 8, _DP)
    out_ref[...] = jnp.concatenate(
        [z[:, s, :] for s in range(8)], axis=-1
    )


def _finish_body(flat_ref, out_ref):
    x = flat_ref[...]
    pieces = [x[:, 16 * s : 16 * s + 12] for s in range(8)]
    y = jnp.concatenate([p[:, None, :] for p in pieces], axis=1)
    out_ref[...] = y.reshape(x.shape[0] * 8, 12)


def kernel(se3, indices):
    V, D = se3.shape
    B = indices.shape[0]

    table_flat = pl.pallas_call(
        _prep_body,
        out_shape=jax.ShapeDtypeStruct((V // 8, 128), jnp.float32),
    )(se3)

    call = _gather_call(V, B)
    rows = call(table_flat.reshape(V, _DP), indices.astype(jnp.int32))

    fin_grid = 16
    fin_blk = (B // 8) // fin_grid
    out = pl.pallas_call(
        _finish_body,
        grid=(fin_grid,),
        in_specs=[pl.BlockSpec((fin_blk, 128), lambda i: (i, 0))],
        out_specs=pl.BlockSpec((fin_blk * 8, D), lambda i: (i, 0)),
        out_shape=jax.ShapeDtypeStruct((B, D), jnp.float32),
    )(rows.reshape(B // 8, 128))
    return out


# speedup vs baseline: 1.2421x; 1.2421x over previous
import functools

import jax
import jax.numpy as jnp
from jax import lax
from jax.experimental import pallas as pl
from jax.experimental.pallas import tpu as pltpu
from jax.experimental.pallas import tpu_sc as plsc

_DP = 16


def _gather_call(V, B):
    info = plsc.get_sparse_core_info()
    NC, NS = info.num_cores, info.num_subcores
    NW = NC * NS
    assert B % (8 * NW) == 0
    b_per_w = B // NW
    mesh = plsc.VectorSubcoreMesh(core_axis_name="c", subcore_axis_name="s")

    chunk = 128
    n_chunks = b_per_w // chunk
    assert n_chunks * chunk == b_per_w

    @functools.partial(
        pl.kernel,
        mesh=mesh,
        out_type=jax.ShapeDtypeStruct((B, _DP), jnp.float32),
        scratch_types=[
            pltpu.VMEM((b_per_w,), jnp.int32),
            pltpu.VMEM((b_per_w, _DP), jnp.float32),
            pltpu.SemaphoreType.DMA,
            pltpu.SemaphoreType.DMA,
        ],
        compiler_params=pltpu.CompilerParams(use_tc_tiling_on_sc=False),
    )
    def k(table_hbm, idx_hbm, out_hbm, idx_v, rows_v, sem_g, sem_w):
        wid = lax.axis_index("s") * NC + lax.axis_index("c")
        base = wid * b_per_w
        pltpu.sync_copy(idx_hbm.at[pl.ds(base, b_per_w)], idx_v)
        gathers = [
            pltpu.async_copy(
                table_hbm.at[idx_v.at[pl.ds(j * chunk, chunk)]],
                rows_v.at[pl.ds(j * chunk, chunk)],
                sem_g,
            )
            for j in range(n_chunks)
        ]
        writes = []
        for j in range(n_chunks):
            gathers[j].wait()
            writes.append(
                pltpu.async_copy(
                    rows_v.at[pl.ds(j * chunk, chunk)],
                    out_hbm.at[pl.ds(base + j * chunk, chunk)],
                    sem_w,
                )
            )
        for w in writes:
            w.wait()

    return k


def _prep_body(se3_ref, out_ref):
    rows, d = se3_ref.shape
    x = se3_ref[...]
    in3 = x.reshape(rows // 8, 8, d)
    acc = None
    for s in range(8):
        xs = in3[:, s, :]
        xq = jnp.pad(xs, ((0, 0), (0, 128 - d)))
        part = pltpu.roll(xq, _DP * s, 1) if s else xq
        acc = part if acc is None else acc + part
    out_ref[...] = acc


def _finish_body(flat_ref, out_ref):
    n, d = out_ref.shape
    x2 = flat_ref[...]
    o3 = out_ref.reshape(n // 8, 8, d)
    for s in range(8):
        rolled = pltpu.roll(x2, (128 - _DP * s) % 128, 1) if s else x2
        o3[:, s, :] = rolled[:, :d]


def kernel(se3, indices):
    V, D = se3.shape
    B = indices.shape[0]

    table2d = pl.pallas_call(
        _prep_body,
        out_shape=jax.ShapeDtypeStruct((V // 8, 128), jnp.float32),
    )(se3)

    call = _gather_call(V, B)
    rows = call(table2d.reshape(V, _DP), indices.astype(jnp.int32))

    fin_grid = 8
    blk = B // fin_grid
    out = pl.pallas_call(
        _finish_body,
        grid=(fin_grid,),
        in_specs=[pl.BlockSpec((blk // 8, 128), lambda i: (i, 0))],
        out_specs=pl.BlockSpec((blk, D), lambda i: (i, 0)),
        out_shape=jax.ShapeDtypeStruct((B, D), jnp.float32),
    )(rows.reshape(B // 8, 128))
    return out


# speedup vs baseline: 1.3352x; 1.0750x over previous
import functools

import jax
import jax.numpy as jnp
from jax import lax
from jax.experimental import pallas as pl
from jax.experimental.pallas import tpu as pltpu
from jax.experimental.pallas import tpu_sc as plsc

_DP = 16


def _gather_call(V, B):
    info = plsc.get_sparse_core_info()
    NC, NS = info.num_cores, info.num_subcores
    NW = NC * NS
    assert B % (8 * NW) == 0
    b_per_w = B // NW
    mesh = plsc.VectorSubcoreMesh(core_axis_name="c", subcore_axis_name="s")

    chunk = 128
    n_chunks = b_per_w // chunk
    assert n_chunks * chunk == b_per_w

    @functools.partial(
        pl.kernel,
        mesh=mesh,
        out_type=jax.ShapeDtypeStruct((B, _DP), jnp.float32),
        scratch_types=[
            pltpu.VMEM((b_per_w,), jnp.int32),
            pltpu.VMEM((b_per_w, _DP), jnp.float32),
            pltpu.SemaphoreType.DMA,
            pltpu.SemaphoreType.DMA,
        ],
        compiler_params=pltpu.CompilerParams(use_tc_tiling_on_sc=False),
    )
    def k(table_hbm, idx_hbm, out_hbm, idx_v, rows_v, sem_g, sem_w):
        wid = lax.axis_index("s") * NC + lax.axis_index("c")
        base = wid * b_per_w
        pltpu.sync_copy(idx_hbm.at[pl.ds(base, b_per_w)], idx_v)
        gathers = [
            pltpu.async_copy(
                table_hbm.at[idx_v.at[pl.ds(j * chunk, chunk)]],
                rows_v.at[pl.ds(j * chunk, chunk)],
                sem_g,
            )
            for j in range(n_chunks)
        ]
        writes = []
        for j in range(n_chunks):
            gathers[j].wait()
            writes.append(
                pltpu.async_copy(
                    rows_v.at[pl.ds(j * chunk, chunk)],
                    out_hbm.at[pl.ds(base + j * chunk, chunk)],
                    sem_w,
                )
            )
        for w in writes:
            w.wait()

    return k


def kernel(se3, indices):
    V, D = se3.shape
    B = indices.shape[0]
    table = jnp.concatenate(
        [se3, jnp.zeros((V, _DP - D), jnp.float32)], axis=1
    )
    call = _gather_call(V, B)
    out = call(table, indices.astype(jnp.int32))
    return out[:, :D]
